# baseline (speedup 1.0000x reference)
import jax
import jax.numpy as jnp
from jax import lax
from jax.experimental import pallas as pl
from jax.experimental.pallas import tpu as pltpu
from jax.experimental.pallas import tpu_sc as plsc

VMIN, VMAX, BINS, WIDTH = 0.0, 1.0, 32, 128

NC, NS, L = 2, 16, 16
NW = NC * NS
ROWS, COLS = 16384, 50
IPW = ROWS // NW
CHUNK = 256
HPW = IPW // CHUNK
NCHUNK = COLS * HPW
NSUB = 4
SUB = CHUNK // NSUB


def _body(xt_hbm, table_hbm, out_hbm, xv, idxg, rows, table_v, table_sh,
          sem_x0, sem_x1, sem_g0, sem_g1, sem_o0, sem_o1):
    sem_x = (sem_x0, sem_x1)
    sem_g = (sem_g0, sem_g1)
    sem_o = (sem_o0, sem_o1)
    sid = lax.axis_index("s")
    wid = sid * NC + lax.axis_index("c")
    ibase = wid * IPW

    @pl.when(sid == 0)
    def _stage():
        pltpu.sync_copy(table_hbm, table_v)
        pltpu.sync_copy(table_v, table_sh)

    plsc.subcore_barrier()

    def offs(g):
        plane = jnp.right_shift(g, 1)
        i0 = ibase + jnp.bitwise_and(g, 1) * CHUNK
        return plane, i0

    def x_copy(g, b):
        plane, i0 = offs(lax.min(g, NCHUNK - 1))
        return pltpu.make_async_copy(
            xt_hbm.at[plane, pl.ds(i0, CHUNK)], xv.at[b], sem_x[b]
        )

    def quantize(b):
        for k in range(CHUNK // L):
            xk = xv[b, pl.ds(k * L, L)]
            qf = jnp.clip(xk * float(BINS), 0.0, float(BINS - 1))
            qi = qf.astype(jnp.int32)
            qi = jnp.where(xk != xk, jnp.full((L,), BINS, jnp.int32), qi)
            idxg[b, k * L // SUB, pl.ds((k * L) % SUB, L)] = qi

    def gathers(b):
        return [
            pltpu.make_async_copy(
                table_sh.at[idxg.at[b].at[m]],
                rows.at[b].at[pl.ds(m * SUB, SUB)],
                sem_g[b],
            )
            for m in range(NSUB)
        ]

    def out_copy(g, b):
        plane, i0 = offs(g)
        return pltpu.make_async_copy(
            rows.at[b], out_hbm.at[pl.ds(plane * ROWS + i0, CHUNK)], sem_o[b]
        )

    def step(g, b, wait_out):
        x_copy(g + 1, b ^ 1).start()
        x_copy(g, b).wait()
        if wait_out:
            out_copy(g - 2, b).wait()
        quantize(b)
        for d in gathers(b):
            d.start()
        for d in gathers(b ^ 1):
            d.wait()
        out_copy(g - 1, b ^ 1).start()

    x_copy(0, 0).start()
    x_copy(0, 0).wait()
    quantize(0)
    for d in gathers(0):
        d.start()
    x_copy(1, 1).start()
    x_copy(1, 1).wait()
    quantize(1)
    for d in gathers(1):
        d.start()
    for d in gathers(0):
        d.wait()
    out_copy(0, 0).start()
    x_copy(2, 0).start()

    def chunk_body(g2, carry):
        step(2 * g2, 0, wait_out=True)
        step(2 * g2 + 1, 1, wait_out=True)
        return carry

    lax.fori_loop(1, NCHUNK // 2, chunk_body, 0)

    for d in gathers(1):
        d.wait()
    out_copy(NCHUNK - 1, 1).start()
    x_copy(NCHUNK - 1, 0).wait()
    out_copy(NCHUNK - 2, 0).wait()
    out_copy(NCHUNK - 1, 1).wait()


def kernel(x, embed_weight):
    mesh = plsc.VectorSubcoreMesh(
        core_axis_name="c", subcore_axis_name="s", num_cores=NC, num_subcores=NS
    )
    out = pl.kernel(
        _body,
        out_type=jax.ShapeDtypeStruct((COLS * ROWS, WIDTH), jnp.float32),
        mesh=mesh,
        compiler_params=pltpu.CompilerParams(
            use_tc_tiling_on_sc=True, needs_layout_passes=False
        ),
        scratch_types=[
            pltpu.VMEM((2, CHUNK), jnp.float32),
            pltpu.VMEM((2, NSUB, SUB), jnp.int32),
            pltpu.VMEM((2, CHUNK, WIDTH), jnp.float32),
            pltpu.VMEM((BINS + 1, WIDTH), jnp.float32),
            pltpu.VMEM_SHARED((BINS + 1, WIDTH), jnp.float32),
            pltpu.SemaphoreType.DMA,
            pltpu.SemaphoreType.DMA,
            pltpu.SemaphoreType.DMA,
            pltpu.SemaphoreType.DMA,
            pltpu.SemaphoreType.DMA,
            pltpu.SemaphoreType.DMA,
        ],
    )(jnp.transpose(x), embed_weight)
    return jnp.transpose(out.reshape(COLS, ROWS, WIDTH), (1, 0, 2))


# speedup vs baseline: 1.0026x; 1.0026x over previous
import jax
import jax.numpy as jnp
from jax import lax
from jax.experimental import pallas as pl
from jax.experimental.pallas import tpu as pltpu
from jax.experimental.pallas import tpu_sc as plsc

VMIN, VMAX, BINS, WIDTH = 0.0, 1.0, 32, 128

NC, NS, L = 2, 16, 16
NW = NC * NS
ROWS, COLS = 16384, 50
IPW = ROWS // NW
CHUNK = 256
HPW = IPW // CHUNK
NCHUNK = COLS * HPW
NSUB = 2
SUB = CHUNK // NSUB


def _body(xt_hbm, table_hbm, out_hbm, xv, idxg, rows, table_v, table_sh,
          sem_x0, sem_x1, sem_g0, sem_g1, sem_o0, sem_o1):
    sem_x = (sem_x0, sem_x1)
    sem_g = (sem_g0, sem_g1)
    sem_o = (sem_o0, sem_o1)
    sid = lax.axis_index("s")
    wid = sid * NC + lax.axis_index("c")
    ibase = wid * IPW

    @pl.when(sid == 0)
    def _stage():
        pltpu.sync_copy(table_hbm, table_v)
        pltpu.sync_copy(table_v, table_sh)

    plsc.subcore_barrier()

    def offs(g):
        plane = jnp.right_shift(g, 1)
        i0 = ibase + jnp.bitwise_and(g, 1) * CHUNK
        return plane, i0

    def x_copy(g, b):
        plane, i0 = offs(lax.min(g, NCHUNK - 1))
        return pltpu.make_async_copy(
            xt_hbm.at[plane, pl.ds(i0, CHUNK)], xv.at[b], sem_x[b]
        )

    def quantize(b):
        for k in range(CHUNK // L):
            xk = xv[b, pl.ds(k * L, L)]
            qf = jnp.clip(xk * float(BINS), 0.0, float(BINS - 1))
            qi = qf.astype(jnp.int32)
            qi = jnp.where(xk != xk, jnp.full((L,), BINS, jnp.int32), qi)
            idxg[b, k * L // SUB, pl.ds((k * L) % SUB, L)] = qi

    def gathers(b):
        return [
            pltpu.make_async_copy(
                table_sh.at[idxg.at[b].at[m]],
                rows.at[b].at[pl.ds(m * SUB, SUB)],
                sem_g[b],
            )
            for m in range(NSUB)
        ]

    def out_copy(g, b):
        plane, i0 = offs(g)
        return pltpu.make_async_copy(
            rows.at[b], out_hbm.at[pl.ds(plane * ROWS + i0, CHUNK)], sem_o[b]
        )

    def step(g, b, wait_out):
        x_copy(g + 1, b ^ 1).start()
        x_copy(g, b).wait()
        if wait_out:
            out_copy(g - 2, b).wait()
        quantize(b)
        for d in gathers(b):
            d.start()
        for d in gathers(b ^ 1):
            d.wait()
        out_copy(g - 1, b ^ 1).start()

    x_copy(0, 0).start()
    x_copy(0, 0).wait()
    quantize(0)
    for d in gathers(0):
        d.start()
    x_copy(1, 1).start()
    x_copy(1, 1).wait()
    quantize(1)
    for d in gathers(1):
        d.start()
    for d in gathers(0):
        d.wait()
    out_copy(0, 0).start()
    x_copy(2, 0).start()

    def chunk_body(g2, carry):
        step(2 * g2, 0, wait_out=True)
        step(2 * g2 + 1, 1, wait_out=True)
        return carry

    lax.fori_loop(1, NCHUNK // 2, chunk_body, 0)

    for d in gathers(1):
        d.wait()
    out_copy(NCHUNK - 1, 1).start()
    x_copy(NCHUNK - 1, 0).wait()
    out_copy(NCHUNK - 2, 0).wait()
    out_copy(NCHUNK - 1, 1).wait()


def kernel(x, embed_weight):
    mesh = plsc.VectorSubcoreMesh(
        core_axis_name="c", subcore_axis_name="s", num_cores=NC, num_subcores=NS
    )
    out = pl.kernel(
        _body,
        out_type=jax.ShapeDtypeStruct((COLS * ROWS, WIDTH), jnp.float32),
        mesh=mesh,
        compiler_params=pltpu.CompilerParams(
            use_tc_tiling_on_sc=True, needs_layout_passes=False
        ),
        scratch_types=[
            pltpu.VMEM((2, CHUNK), jnp.float32),
            pltpu.VMEM((2, NSUB, SUB), jnp.int32),
            pltpu.VMEM((2, CHUNK, WIDTH), jnp.float32),
            pltpu.VMEM((BINS + 1, WIDTH), jnp.float32),
            pltpu.VMEM_SHARED((BINS + 1, WIDTH), jnp.float32),
            pltpu.SemaphoreType.DMA,
            pltpu.SemaphoreType.DMA,
            pltpu.SemaphoreType.DMA,
            pltpu.SemaphoreType.DMA,
            pltpu.SemaphoreType.DMA,
            pltpu.SemaphoreType.DMA,
        ],
    )(jnp.transpose(x), embed_weight)
    return jnp.transpose(out.reshape(COLS, ROWS, WIDTH), (1, 0, 2))
